# Initial kernel scaffold; baseline (speedup 1.0000x reference)
import jax
import jax.numpy as jnp
from jax.experimental import pallas as pl


def kernel(maps_1, maps_2):
    raise NotImplementedError("write your pallas kernel here")


# speedup vs baseline: 1.7504x; 1.7504x over previous
import jax
import jax.numpy as jnp
from jax import lax
from jax.experimental import pallas as pl

_B, _C, _H, _W = 64, 384, 14, 14
_N = _H * _W
_INV_COEFF, _STD_COEFF, _COV_COEFF = 25.0, 25.0, 1.0
_EPS = 1e-05
_GAMMA = 1.0
_SB = 8
_NB = 28

_PREC = lax.Precision.HIGHEST


def _phase1_body(m1_ref, m2_ref, m1t_ref, m2t_ref, nn1t_ref, nn2t_ref,
                 inv_ref):
    x = m1_ref[...]
    y = m2_ref[...]
    g = lax.dot_general(x, y, (((2,), (2,)), ((0,), (0,))),
                        preferred_element_type=jnp.float32, precision=_PREC)
    x2 = jnp.sum(x * x, axis=2)
    y2 = jnp.sum(y * y, axis=2)
    d2 = x2[:, :, None] - 2.0 * g + y2[:, None, :]
    col = lax.broadcasted_iota(jnp.int32, (_SB, _N, _N), 2)
    min1 = jnp.min(d2, axis=2, keepdims=True)
    idx1 = jnp.min(jnp.where(d2 <= min1, col, _N), axis=2)
    row = lax.broadcasted_iota(jnp.int32, (_SB, _N, _N), 1)
    big = jnp.where(d2 <= jnp.min(d2, axis=1, keepdims=True), row, _N)
    idx2 = jnp.min(big, axis=1)
    oh1 = (col == idx1[:, :, None]).astype(jnp.float32)
    oh2 = (col == idx2[:, :, None]).astype(jnp.float32)
    nn1 = lax.dot_general(oh1, y, (((2,), (1,)), ((0,), (0,))),
                          preferred_element_type=jnp.float32, precision=_PREC)
    nn2 = lax.dot_general(oh2, x, (((2,), (1,)), ((0,), (0,))),
                          preferred_element_type=jnp.float32, precision=_PREC)
    d1 = x - nn1
    dd2 = y - nn2
    inv_part = jnp.sum(d1 * d1, axis=(1, 2)) + jnp.sum(dd2 * dd2, axis=(1, 2))
    inv_ref[0] = inv_part[None, :]
    for s in range(_SB):
        m1t_ref[:, s, :] = x[s]
        m2t_ref[:, s, :] = y[s]
        nn1t_ref[:, s, :] = nn1[s]
        nn2t_ref[:, s, :] = nn2[s]


def _stack_stats(s):
    mu = jnp.mean(s, axis=1, keepdims=True)
    a = s - mu
    var = jnp.sum(a * a, axis=1) / (_B - 1)
    stdsum = jnp.sum(jnp.maximum(_GAMMA - jnp.sqrt(var + _EPS), 0.0))
    gram = lax.dot_general(a, a, (((2,), (2,)), ((0,), (0,))),
                           preferred_element_type=jnp.float32,
                           precision=_PREC)
    covsum = (jnp.sum(gram * gram) / ((_B - 1) ** 2)
              - jnp.sum(var * var))
    return stdsum, covsum


def _phase2_body(m1t_ref, m2t_ref, nn1t_ref, nn2t_ref, std_ref, cov_ref):
    i = pl.program_id(0)

    @pl.when(i == 0)
    def _init():
        std_ref[...] = jnp.zeros_like(std_ref)
        cov_ref[...] = jnp.zeros_like(cov_ref)

    stdsum = 0.0
    covsum = 0.0
    for ref in (m1t_ref, m2t_ref, nn1t_ref, nn2t_ref):
        ss, cs = _stack_stats(ref[...])
        stdsum += ss
        covsum += cs
    std_ref[...] += jnp.full(std_ref.shape, stdsum, jnp.float32)
    cov_ref[...] += jnp.full(cov_ref.shape, covsum, jnp.float32)


def _caevl(m1, m2):
    tshape = jax.ShapeDtypeStruct((_N, _B, _C), jnp.float32)
    tspec = pl.BlockSpec((_N, _SB, _C), lambda g: (0, g, 0))
    m1t, m2t, nn1t, nn2t, o_inv = pl.pallas_call(
        _phase1_body,
        grid=(_B // _SB,),
        in_specs=[pl.BlockSpec((_SB, _N, _C), lambda g: (g, 0, 0)),
                  pl.BlockSpec((_SB, _N, _C), lambda g: (g, 0, 0))],
        out_specs=[tspec, tspec, tspec, tspec,
                   pl.BlockSpec((1, 1, _SB), lambda g: (g, 0, 0))],
        out_shape=[tshape, tshape, tshape, tshape,
                   jax.ShapeDtypeStruct((_B // _SB, 1, _SB), jnp.float32)],
    )(m1, m2)

    o_std, o_cov = pl.pallas_call(
        _phase2_body,
        grid=(_N // _NB,),
        in_specs=[pl.BlockSpec((_NB, _B, _C), lambda i: (i, 0, 0))] * 4,
        out_specs=[pl.BlockSpec((1, 128), lambda i: (0, 0)),
                   pl.BlockSpec((1, 128), lambda i: (0, 0))],
        out_shape=[jax.ShapeDtypeStruct((1, 128), jnp.float32),
                   jax.ShapeDtypeStruct((1, 128), jnp.float32)],
    )(m1t, m2t, nn1t, nn2t)

    inv = (_INV_COEFF / 2.0) * o_inv.reshape(_B) / (_N * _C)
    std = (_STD_COEFF / 4.0) * o_std[0, 0] / (_N * _C)
    cov = (_COV_COEFF / (4.0 * _C)) * o_cov[0, 0] / _N
    return inv + std + cov


def kernel(maps_1, maps_2):
    m1 = jnp.transpose(maps_1, (0, 2, 3, 1)).reshape(_B, _N, _C)
    m2 = jnp.transpose(maps_2, (0, 2, 3, 1)).reshape(_B, _N, _C)
    return _caevl(m1, m2)


# speedup vs baseline: 3.2771x; 1.8722x over previous
import jax
import jax.numpy as jnp
from jax import lax
from jax.experimental import pallas as pl

_B, _C, _H, _W = 64, 384, 14, 14
_N = _H * _W
_INV_COEFF, _STD_COEFF, _COV_COEFF = 25.0, 25.0, 1.0
_EPS = 1e-05
_GAMMA = 1.0
_SB = 8
_NB = 49

_PREC = lax.Precision.DEFAULT


def _phase1_body(m1_ref, m2_ref, m1t_ref, m2t_ref, nn1t_ref, nn2t_ref,
                 inv_ref):
    x = m1_ref[...]
    y = m2_ref[...]
    g = lax.dot_general(x, y, (((2,), (2,)), ((0,), (0,))),
                        preferred_element_type=jnp.float32, precision=_PREC)
    x2 = jnp.sum(x * x, axis=2)
    y2 = jnp.sum(y * y, axis=2)
    d2 = x2[:, :, None] - 2.0 * g + y2[:, None, :]
    col = lax.broadcasted_iota(jnp.int32, (_SB, _N, _N), 2)
    min1 = jnp.min(d2, axis=2, keepdims=True)
    idx1 = jnp.min(jnp.where(d2 <= min1, col, _N), axis=2)
    row = lax.broadcasted_iota(jnp.int32, (_SB, _N, _N), 1)
    big = jnp.where(d2 <= jnp.min(d2, axis=1, keepdims=True), row, _N)
    idx2 = jnp.min(big, axis=1)
    oh1 = (col == idx1[:, :, None]).astype(jnp.float32)
    oh2 = (col == idx2[:, :, None]).astype(jnp.float32)
    nn1 = lax.dot_general(oh1, y, (((2,), (1,)), ((0,), (0,))),
                          preferred_element_type=jnp.float32, precision=_PREC)
    nn2 = lax.dot_general(oh2, x, (((2,), (1,)), ((0,), (0,))),
                          preferred_element_type=jnp.float32, precision=_PREC)
    d1 = x - nn1
    dd2 = y - nn2
    inv_part = jnp.sum(d1 * d1, axis=(1, 2)) + jnp.sum(dd2 * dd2, axis=(1, 2))
    inv_ref[0] = inv_part[None, :]
    for s in range(_SB):
        m1t_ref[:, s, :] = x[s]
        m2t_ref[:, s, :] = y[s]
        nn1t_ref[:, s, :] = nn1[s]
        nn2t_ref[:, s, :] = nn2[s]


def _stack_stats(s):
    mu = jnp.mean(s, axis=1, keepdims=True)
    a = s - mu
    var = jnp.sum(a * a, axis=1) / (_B - 1)
    stdsum = jnp.sum(jnp.maximum(_GAMMA - jnp.sqrt(var + _EPS), 0.0))
    gram = lax.dot_general(a, a, (((2,), (2,)), ((0,), (0,))),
                           preferred_element_type=jnp.float32,
                           precision=_PREC)
    covsum = (jnp.sum(gram * gram) / ((_B - 1) ** 2)
              - jnp.sum(var * var))
    return stdsum, covsum


def _phase2_body(m1t_ref, m2t_ref, nn1t_ref, nn2t_ref, std_ref, cov_ref):
    i = pl.program_id(0)

    @pl.when(i == 0)
    def _init():
        std_ref[...] = jnp.zeros_like(std_ref)
        cov_ref[...] = jnp.zeros_like(cov_ref)

    stdsum = 0.0
    covsum = 0.0
    for ref in (m1t_ref, m2t_ref, nn1t_ref, nn2t_ref):
        ss, cs = _stack_stats(ref[...])
        stdsum += ss
        covsum += cs
    std_ref[...] += jnp.full(std_ref.shape, stdsum, jnp.float32)
    cov_ref[...] += jnp.full(cov_ref.shape, covsum, jnp.float32)


def _caevl(m1, m2):
    tshape = jax.ShapeDtypeStruct((_N, _B, _C), jnp.float32)
    tspec = pl.BlockSpec((_N, _SB, _C), lambda g: (0, g, 0))
    m1t, m2t, nn1t, nn2t, o_inv = pl.pallas_call(
        _phase1_body,
        grid=(_B // _SB,),
        in_specs=[pl.BlockSpec((_SB, _N, _C), lambda g: (g, 0, 0)),
                  pl.BlockSpec((_SB, _N, _C), lambda g: (g, 0, 0))],
        out_specs=[tspec, tspec, tspec, tspec,
                   pl.BlockSpec((1, 1, _SB), lambda g: (g, 0, 0))],
        out_shape=[tshape, tshape, tshape, tshape,
                   jax.ShapeDtypeStruct((_B // _SB, 1, _SB), jnp.float32)],
    )(m1, m2)

    o_std, o_cov = pl.pallas_call(
        _phase2_body,
        grid=(_N // _NB,),
        in_specs=[pl.BlockSpec((_NB, _B, _C), lambda i: (i, 0, 0))] * 4,
        out_specs=[pl.BlockSpec((1, 128), lambda i: (0, 0)),
                   pl.BlockSpec((1, 128), lambda i: (0, 0))],
        out_shape=[jax.ShapeDtypeStruct((1, 128), jnp.float32),
                   jax.ShapeDtypeStruct((1, 128), jnp.float32)],
    )(m1t, m2t, nn1t, nn2t)

    inv = (_INV_COEFF / 2.0) * o_inv.reshape(_B) / (_N * _C)
    std = (_STD_COEFF / 4.0) * o_std[0, 0] / (_N * _C)
    cov = (_COV_COEFF / (4.0 * _C)) * o_cov[0, 0] / _N
    return inv + std + cov


def kernel(maps_1, maps_2):
    m1 = jnp.transpose(maps_1, (0, 2, 3, 1)).reshape(_B, _N, _C)
    m2 = jnp.transpose(maps_2, (0, 2, 3, 1)).reshape(_B, _N, _C)
    return _caevl(m1, m2)
